# Initial kernel scaffold; baseline (speedup 1.0000x reference)
import jax
import jax.numpy as jnp
from jax.experimental import pallas as pl


def kernel(x, table):
    raise NotImplementedError("write your pallas kernel here")


# speedup vs baseline: 1.6036x; 1.6036x over previous
import jax
import jax.numpy as jnp
from jax import lax
from jax.experimental import pallas as pl
from jax.experimental.pallas import tpu as pltpu
from jax.experimental.pallas import tpu_sc as plsc

NUM_CORES = 2
NUM_SUBCORES = 16
NUM_WORKERS = NUM_CORES * NUM_SUBCORES
CHUNK = 512


def kernel(x, table):
    batch, hist = x.shape
    vocab, d = table.shape
    n = batch * hist
    idx = x.reshape(n).astype(jnp.int32)
    table_p = jnp.concatenate([table, jnp.zeros_like(table)], axis=1)

    per_worker = n // NUM_WORKERS
    n_chunks = per_worker // CHUNK
    assert per_worker % CHUNK == 0

    mesh = plsc.VectorSubcoreMesh(core_axis_name="c", subcore_axis_name="s")

    @pl.kernel(
        out_type=jax.ShapeDtypeStruct((n, 2 * d), table.dtype),
        mesh=mesh,
        scratch_types=[
            pltpu.VMEM((CHUNK,), jnp.int32),
            pltpu.VMEM((CHUNK, 2 * d), jnp.float32),
            pltpu.SemaphoreType.DMA,
        ],
    )
    def gather_kernel(table_hbm, idx_hbm, out_hbm, idx_v, rows_v, sem):
        wid = lax.axis_index("s") * NUM_CORES + lax.axis_index("c")
        base = wid * per_worker

        @pl.loop(0, n_chunks)
        def _(c):
            off = base + c * CHUNK
            pltpu.sync_copy(idx_hbm.at[pl.ds(off, CHUNK)], idx_v)
            pltpu.async_copy(table_hbm.at[idx_v], rows_v, sem).wait()
            pltpu.sync_copy(rows_v, out_hbm.at[pl.ds(off, CHUNK)])

    out = gather_kernel(table_p, idx)
    return out[:, :d].reshape(batch, hist, d)


# speedup vs baseline: 1.8238x; 1.1373x over previous
import jax
import jax.numpy as jnp
from jax import lax
from jax.experimental import pallas as pl
from jax.experimental.pallas import tpu as pltpu
from jax.experimental.pallas import tpu_sc as plsc

NUM_CORES = 2
NUM_SUBCORES = 16
NUM_WORKERS = NUM_CORES * NUM_SUBCORES
NB = 8
LANES = 16


def kernel(x, table):
    batch, hist = x.shape
    vocab, d = table.shape
    n = batch * hist
    idx = x.reshape(n).astype(jnp.int32)
    table_p = jnp.concatenate([table, jnp.zeros_like(table)], axis=1)

    rows_per_worker = batch // NUM_WORKERS
    n_chunks = rows_per_worker // NB
    chunk = NB * hist
    assert batch % NUM_WORKERS == 0 and rows_per_worker % NB == 0

    mesh = plsc.VectorSubcoreMesh(core_axis_name="c", subcore_axis_name="s")

    @pl.kernel(
        out_type=jax.ShapeDtypeStruct((batch, hist, d), table.dtype),
        mesh=mesh,
        scratch_types=[
            pltpu.VMEM((chunk,), jnp.int32),
            pltpu.VMEM((chunk, 2 * d), jnp.float32),
            pltpu.VMEM((chunk, d), jnp.float32),
            pltpu.SemaphoreType.DMA,
        ],
    )
    def gather_kernel(table_hbm, idx_hbm, out_hbm, idx_v, rows_v, cmp_v, sem):
        wid = lax.axis_index("s") * NUM_CORES + lax.axis_index("c")
        row_base = wid * rows_per_worker

        @pl.loop(0, n_chunks)
        def _(c):
            b0 = row_base + c * NB
            pltpu.sync_copy(idx_hbm.at[pl.ds(b0 * hist, chunk)], idx_v)
            pltpu.async_copy(table_hbm.at[idx_v], rows_v, sem).wait()

            @pl.loop(0, chunk)
            def _(r):
                for k in range(d // LANES):
                    sl = pl.ds(k * LANES, LANES)
                    cmp_v[pl.ds(r, 1), sl] = rows_v[pl.ds(r, 1), sl]

            copies = [
                pltpu.async_copy(
                    cmp_v.at[pl.ds(j * hist, hist)],
                    out_hbm.at[b0 + j],
                    sem,
                )
                for j in range(NB)
            ]
            for cp in copies:
                cp.wait()

    out = gather_kernel(table_p, idx)
    return out
